# Initial kernel scaffold; baseline (speedup 1.0000x reference)
import functools

import jax
import jax.numpy as jnp
from jax import lax
from jax.experimental import pallas as pl
from jax.experimental.pallas import tpu as pltpu
from jax.experimental.pallas import tpu_sc as plsc

B = 512
D = 128
N_ATOMS = 10000
N_BOND_ROWS = 320000
N_BONDS = N_BOND_ROWS // 2

NC = 2
NS = 16
NW = NC * NS

CHUNK = 128
ATOM_PT = 384
BOND_PT = 5120
ATOM_TOT = ATOM_PT * NW
BOND_TOT = BOND_PT * NW
NA_CHUNKS = ATOM_PT // CHUNK
NB_CHUNKS = BOND_PT // CHUNK

DUMMY = B
ACC_ROWS = 528

_mesh = plsc.VectorSubcoreMesh(core_axis_name="c", subcore_axis_name="s")


@functools.partial(
    pl.kernel,
    out_type=[
        jax.ShapeDtypeStruct((NC, B, D), jnp.float32),
        jax.ShapeDtypeStruct((NC, B, D), jnp.float32),
    ],
    mesh=_mesh,
    scratch_types=[
        pltpu.VMEM((NA_CHUNKS, CHUNK), jnp.int32),
        pltpu.VMEM((NA_CHUNKS, CHUNK), jnp.int32),
        pltpu.VMEM((NB_CHUNKS, CHUNK), jnp.int32),
        pltpu.VMEM((NB_CHUNKS, CHUNK), jnp.int32),
        pltpu.VMEM((CHUNK, D), jnp.float32),
        pltpu.VMEM((33, D), jnp.float32),
        pltpu.VMEM_SHARED((ACC_ROWS, D), jnp.float32),
        pltpu.VMEM_SHARED((ACC_ROWS, D), jnp.float32),
        pltpu.SemaphoreType.DMA,
    ],
)
def _sc_pool(atom_hbm, bond_hbm, aid_hbm, bid_hbm, pa_hbm, pb_hbm,
             aidx_v, aids_v, bidx_v, bids_v, buf, zbuf, acc_a, acc_b, sem):
    cid = lax.axis_index("c")
    sid = lax.axis_index("s")
    wid = cid * NS + sid

    zvec = jnp.zeros((16,), jnp.float32)
    for r in range(33):
        for g in range(D // 16):
            zbuf[r, pl.ds(g * 16, 16)] = zvec
    pltpu.sync_copy(zbuf, acc_a.at[pl.ds(sid * 33, 33)])
    pltpu.sync_copy(zbuf, acc_b.at[pl.ds(sid * 33, 33)])

    pltpu.sync_copy(aid_hbm.at[pl.ds(wid * NA_CHUNKS, NA_CHUNKS)], aids_v)
    pltpu.sync_copy(bid_hbm.at[pl.ds(wid * NB_CHUNKS, NB_CHUNKS)], bids_v)

    iota = lax.iota(jnp.int32, 16)
    abase = wid * ATOM_PT
    bbase = wid * BOND_PT

    def build_a(j, carry):
        for g in range(CHUNK // 16):
            aidx_v[j, pl.ds(g * 16, 16)] = jnp.minimum(
                abase + j * CHUNK + g * 16 + iota, N_ATOMS - 1)
        return carry

    def build_b(j, carry):
        for g in range(CHUNK // 16):
            bidx_v[j, pl.ds(g * 16, 16)] = jnp.minimum(
                (bbase + j * CHUNK + g * 16 + iota) * 2, N_BOND_ROWS - 2)
        return carry

    lax.fori_loop(0, NA_CHUNKS, build_a, 0)
    lax.fori_loop(0, NB_CHUNKS, build_b, 0)

    plsc.subcore_barrier()

    def atom_step(j, carry):
        pltpu.async_copy(atom_hbm.at[aidx_v.at[j]], buf, sem).wait()
        pltpu.sync_copy(buf, acc_a.at[aids_v.at[j]], add=True)
        return carry

    def bond_step(j, carry):
        pltpu.async_copy(bond_hbm.at[bidx_v.at[j]], buf, sem).wait()
        pltpu.sync_copy(buf, acc_b.at[bids_v.at[j]], add=True)
        return carry

    lax.fori_loop(0, NA_CHUNKS, atom_step, 0)
    lax.fori_loop(0, NB_CHUNKS, bond_step, 0)

    plsc.subcore_barrier()

    pltpu.sync_copy(acc_a.at[pl.ds(sid * 32, 32)], zbuf.at[pl.ds(0, 32)])
    pltpu.sync_copy(zbuf.at[pl.ds(0, 32)], pa_hbm.at[cid, pl.ds(sid * 32, 32)])
    pltpu.sync_copy(acc_b.at[pl.ds(sid * 32, 32)], zbuf.at[pl.ds(0, 32)])
    pltpu.sync_copy(zbuf.at[pl.ds(0, 32)], pb_hbm.at[cid, pl.ds(sid * 32, 32)])


def _combine_body(pa_ref, pb_ref, g_ref, o_ref):
    o_ref[:, 0:D] = pa_ref[0] + pa_ref[1]
    o_ref[:, D:2 * D] = pb_ref[0] + pb_ref[1]
    o_ref[:, 2 * D:3 * D] = g_ref[:]


_combine = pl.pallas_call(
    _combine_body,
    out_shape=jax.ShapeDtypeStruct((B, 3 * D), jnp.float32),
)


def kernel(atom_feats, bond_feats, global_feats, atom_segment_ids, bond_segment_ids):
    aid = atom_segment_ids.astype(jnp.int32)
    bid = bond_segment_ids.astype(jnp.int32)
    aid_p = jnp.concatenate(
        [aid, jnp.full((ATOM_TOT - N_ATOMS,), DUMMY, jnp.int32)]
    ).reshape(ATOM_TOT // CHUNK, CHUNK)
    bid_p = jnp.concatenate(
        [bid, jnp.full((BOND_TOT - N_BONDS,), DUMMY, jnp.int32)]
    ).reshape(BOND_TOT // CHUNK, CHUNK)
    pa, pb = _sc_pool(atom_feats, bond_feats, aid_p, bid_p)
    return _combine(pa, pb, global_feats)


# speedup vs baseline: 2.4887x; 2.4887x over previous
import functools

import jax
import jax.numpy as jnp
from jax import lax
from jax.experimental import pallas as pl
from jax.experimental.pallas import tpu as pltpu
from jax.experimental.pallas import tpu_sc as plsc

B = 512
D = 128
N_ATOMS = 10000
N_BOND_ROWS = 320000
N_BONDS = N_BOND_ROWS // 2

NC = 2
NS = 16
NW = NC * NS

CHUNK = 128
ATOM_PT = 384
BOND_PT = 5120
ATOM_TOT = ATOM_PT * NW
BOND_TOT = BOND_PT * NW
NA_CHUNKS = ATOM_PT // CHUNK
NB_CHUNKS = BOND_PT // CHUNK

DUMMY = B
ACC_ROWS = 528

_mesh = plsc.VectorSubcoreMesh(core_axis_name="c", subcore_axis_name="s")


@functools.partial(
    pl.kernel,
    out_type=[
        jax.ShapeDtypeStruct((NC, B, D), jnp.float32),
        jax.ShapeDtypeStruct((NC, B, D), jnp.float32),
    ],
    mesh=_mesh,
    scratch_types=[
        pltpu.VMEM((NA_CHUNKS, CHUNK), jnp.int32),
        pltpu.VMEM((NA_CHUNKS, CHUNK), jnp.int32),
        pltpu.VMEM((NB_CHUNKS, CHUNK), jnp.int32),
        pltpu.VMEM((NB_CHUNKS, CHUNK), jnp.int32),
        pltpu.VMEM((CHUNK, D), jnp.float32),
        pltpu.VMEM((33, D), jnp.float32),
        pltpu.VMEM_SHARED((ACC_ROWS, D), jnp.float32),
        pltpu.VMEM_SHARED((ACC_ROWS, D), jnp.float32),
        pltpu.SemaphoreType.DMA,
    ],
)
def _sc_pool(atom_hbm, bond_hbm, aid_hbm, bid_hbm, pa_hbm, pb_hbm,
             aidx_v, aids_v, bidx_v, bids_v, buf, zbuf, acc_a, acc_b, sem):
    cid = lax.axis_index("c")
    sid = lax.axis_index("s")
    wid = cid * NS + sid

    zvec = jnp.zeros((16,), jnp.float32)
    for r in range(33):
        for g in range(D // 16):
            zbuf[r, pl.ds(g * 16, 16)] = zvec
    pltpu.sync_copy(zbuf, acc_a.at[pl.ds(sid * 33, 33)])
    pltpu.sync_copy(zbuf, acc_b.at[pl.ds(sid * 33, 33)])

    pltpu.sync_copy(aid_hbm.at[wid], aids_v)
    pltpu.sync_copy(bid_hbm.at[wid], bids_v)

    iota = lax.iota(jnp.int32, 16)
    abase = wid * ATOM_PT
    bbase = wid * BOND_PT

    def build_a(j, carry):
        for g in range(CHUNK // 16):
            aidx_v[j, pl.ds(g * 16, 16)] = jnp.minimum(
                abase + j * CHUNK + g * 16 + iota, N_ATOMS - 1)
        return carry

    def build_b(j, carry):
        for g in range(CHUNK // 16):
            bidx_v[j, pl.ds(g * 16, 16)] = jnp.minimum(
                (bbase + j * CHUNK + g * 16 + iota) * 2, N_BOND_ROWS - 2)
        return carry

    lax.fori_loop(0, NA_CHUNKS, build_a, 0)
    lax.fori_loop(0, NB_CHUNKS, build_b, 0)

    plsc.subcore_barrier()

    def atom_step(j, carry):
        pltpu.async_copy(atom_hbm.at[aidx_v.at[j]], buf, sem).wait()
        pltpu.sync_copy(buf, acc_a.at[aids_v.at[j]], add=True)
        return carry

    def bond_step(j, carry):
        pltpu.async_copy(bond_hbm.at[bidx_v.at[j]], buf, sem).wait()
        pltpu.sync_copy(buf, acc_b.at[bids_v.at[j]], add=True)
        return carry

    lax.fori_loop(0, NA_CHUNKS, atom_step, 0)
    lax.fori_loop(0, NB_CHUNKS, bond_step, 0)

    plsc.subcore_barrier()

    pltpu.sync_copy(acc_a.at[pl.ds(sid * 32, 32)], zbuf.at[pl.ds(0, 32)])
    pltpu.sync_copy(zbuf.at[pl.ds(0, 32)], pa_hbm.at[cid, pl.ds(sid * 32, 32)])
    pltpu.sync_copy(acc_b.at[pl.ds(sid * 32, 32)], zbuf.at[pl.ds(0, 32)])
    pltpu.sync_copy(zbuf.at[pl.ds(0, 32)], pb_hbm.at[cid, pl.ds(sid * 32, 32)])


def _combine_body(pa_ref, pb_ref, g_ref, o_ref):
    o_ref[:, 0:D] = pa_ref[0] + pa_ref[1]
    o_ref[:, D:2 * D] = pb_ref[0] + pb_ref[1]
    o_ref[:, 2 * D:3 * D] = g_ref[:]


_combine = pl.pallas_call(
    _combine_body,
    out_shape=jax.ShapeDtypeStruct((B, 3 * D), jnp.float32),
)


def kernel(atom_feats, bond_feats, global_feats, atom_segment_ids, bond_segment_ids):
    aid = atom_segment_ids.astype(jnp.int32)
    bid = bond_segment_ids.astype(jnp.int32)
    aid_p = jnp.concatenate(
        [aid, jnp.full((ATOM_TOT - N_ATOMS,), DUMMY, jnp.int32)]
    ).reshape(NW, NA_CHUNKS, CHUNK)
    bid_p = jnp.concatenate(
        [bid, jnp.full((BOND_TOT - N_BONDS,), DUMMY, jnp.int32)]
    ).reshape(NW, NB_CHUNKS, CHUNK)
    pa, pb = _sc_pool(atom_feats, bond_feats, aid_p, bid_p)
    return _combine(pa, pb, global_feats)


# speedup vs baseline: 2.7398x; 1.1009x over previous
import functools

import jax
import jax.numpy as jnp
from jax import lax
from jax.experimental import pallas as pl
from jax.experimental.pallas import tpu as pltpu
from jax.experimental.pallas import tpu_sc as plsc

B = 512
D = 128
N_ATOMS = 10000
N_BOND_ROWS = 320000
N_BONDS = N_BOND_ROWS // 2

NC = 2
NS = 16
NW = NC * NS

CHUNK = 128
ATOM_PT = 384
BOND_PT = 5120
ATOM_TOT = ATOM_PT * NW
BOND_TOT = BOND_PT * NW
NA_CHUNKS = ATOM_PT // CHUNK
NB_CHUNKS = BOND_PT // CHUNK

DUMMY = B
ACC_ROWS = 528

_mesh = plsc.VectorSubcoreMesh(core_axis_name="c", subcore_axis_name="s")


@functools.partial(
    pl.kernel,
    out_type=[
        jax.ShapeDtypeStruct((NC, B, D), jnp.float32),
        jax.ShapeDtypeStruct((NC, B, D), jnp.float32),
    ],
    mesh=_mesh,
    scratch_types=[
        pltpu.VMEM((NA_CHUNKS, CHUNK), jnp.int32),
        pltpu.VMEM((NA_CHUNKS, CHUNK), jnp.int32),
        pltpu.VMEM((NB_CHUNKS, CHUNK), jnp.int32),
        pltpu.VMEM((NB_CHUNKS, CHUNK), jnp.int32),
        pltpu.VMEM((CHUNK, D), jnp.float32),
        pltpu.VMEM((CHUNK, D), jnp.float32),
        pltpu.VMEM((33, D), jnp.float32),
        pltpu.VMEM_SHARED((ACC_ROWS, D), jnp.float32),
        pltpu.VMEM_SHARED((ACC_ROWS, D), jnp.float32),
        pltpu.SemaphoreType.DMA,
        pltpu.SemaphoreType.DMA,
    ],
)
def _sc_pool(atom_hbm, bond_hbm, aid_hbm, bid_hbm, pa_hbm, pb_hbm,
             aidx_v, aids_v, bidx_v, bids_v, buf0, buf1, zbuf, acc_a, acc_b,
             semA, semB):
    cid = lax.axis_index("c")
    sid = lax.axis_index("s")
    wid = cid * NS + sid

    zvec = jnp.zeros((16,), jnp.float32)
    for r in range(33):
        for g in range(D // 16):
            zbuf[r, pl.ds(g * 16, 16)] = zvec
    pltpu.sync_copy(zbuf, acc_a.at[pl.ds(sid * 33, 33)])
    pltpu.sync_copy(zbuf, acc_b.at[pl.ds(sid * 33, 33)])

    pltpu.sync_copy(aid_hbm.at[wid], aids_v)
    pltpu.sync_copy(bid_hbm.at[wid], bids_v)

    iota = lax.iota(jnp.int32, 16)
    abase = wid * ATOM_PT
    bbase = wid * BOND_PT

    def build_a(j, carry):
        for g in range(CHUNK // 16):
            aidx_v[j, pl.ds(g * 16, 16)] = jnp.minimum(
                abase + j * CHUNK + g * 16 + iota, N_ATOMS - 1)
        return carry

    def build_b(j, carry):
        for g in range(CHUNK // 16):
            bidx_v[j, pl.ds(g * 16, 16)] = jnp.minimum(
                (bbase + j * CHUNK + g * 16 + iota) * 2, N_BOND_ROWS - 2)
        return carry

    lax.fori_loop(0, NA_CHUNKS, build_a, 0)
    lax.fori_loop(0, NB_CHUNKS, build_b, 0)

    plsc.subcore_barrier()

    a0 = pltpu.async_copy(atom_hbm.at[aidx_v.at[0]], buf0, semA)
    a1 = pltpu.async_copy(atom_hbm.at[aidx_v.at[1]], buf1, semB)
    a0.wait()
    pltpu.sync_copy(buf0, acc_a.at[aids_v.at[0]], add=True)
    a2 = pltpu.async_copy(atom_hbm.at[aidx_v.at[2]], buf0, semA)
    a1.wait()
    pltpu.sync_copy(buf1, acc_a.at[aids_v.at[1]], add=True)
    b1 = pltpu.async_copy(bond_hbm.at[bidx_v.at[1]], buf1, semB)
    a2.wait()
    pltpu.sync_copy(buf0, acc_a.at[aids_v.at[2]], add=True)
    pltpu.async_copy(bond_hbm.at[bidx_v.at[0]], buf0, semA)

    def bond_pair(p, carry):
        j = 2 * p
        pltpu.make_async_copy(bond_hbm.at[bidx_v.at[j]], buf0, semA).wait()
        pltpu.sync_copy(buf0, acc_b.at[bids_v.at[j]], add=True)
        pltpu.async_copy(bond_hbm.at[bidx_v.at[j + 2]], buf0, semA)
        pltpu.make_async_copy(bond_hbm.at[bidx_v.at[j + 1]], buf1, semB).wait()
        pltpu.sync_copy(buf1, acc_b.at[bids_v.at[j + 1]], add=True)
        pltpu.async_copy(bond_hbm.at[bidx_v.at[j + 3]], buf1, semB)
        return carry

    lax.fori_loop(0, NB_CHUNKS // 2 - 1, bond_pair, 0)
    j = NB_CHUNKS - 2
    pltpu.make_async_copy(bond_hbm.at[bidx_v.at[j]], buf0, semA).wait()
    pltpu.sync_copy(buf0, acc_b.at[bids_v.at[j]], add=True)
    pltpu.make_async_copy(bond_hbm.at[bidx_v.at[j + 1]], buf1, semB).wait()
    pltpu.sync_copy(buf1, acc_b.at[bids_v.at[j + 1]], add=True)

    plsc.subcore_barrier()

    pltpu.sync_copy(acc_a.at[pl.ds(sid * 32, 32)], zbuf.at[pl.ds(0, 32)])
    pltpu.sync_copy(zbuf.at[pl.ds(0, 32)], pa_hbm.at[cid, pl.ds(sid * 32, 32)])
    pltpu.sync_copy(acc_b.at[pl.ds(sid * 32, 32)], zbuf.at[pl.ds(0, 32)])
    pltpu.sync_copy(zbuf.at[pl.ds(0, 32)], pb_hbm.at[cid, pl.ds(sid * 32, 32)])


def _combine_body(pa_ref, pb_ref, g_ref, o_ref):
    o_ref[:, 0:D] = pa_ref[0] + pa_ref[1]
    o_ref[:, D:2 * D] = pb_ref[0] + pb_ref[1]
    o_ref[:, 2 * D:3 * D] = g_ref[:]


_combine = pl.pallas_call(
    _combine_body,
    out_shape=jax.ShapeDtypeStruct((B, 3 * D), jnp.float32),
)


def kernel(atom_feats, bond_feats, global_feats, atom_segment_ids, bond_segment_ids):
    aid = atom_segment_ids.astype(jnp.int32)
    bid = bond_segment_ids.astype(jnp.int32)
    aid_p = jnp.concatenate(
        [aid, jnp.full((ATOM_TOT - N_ATOMS,), DUMMY, jnp.int32)]
    ).reshape(NW, NA_CHUNKS, CHUNK)
    bid_p = jnp.concatenate(
        [bid, jnp.full((BOND_TOT - N_BONDS,), DUMMY, jnp.int32)]
    ).reshape(NW, NB_CHUNKS, CHUNK)
    pa, pb = _sc_pool(atom_feats, bond_feats, aid_p, bid_p)
    return _combine(pa, pb, global_feats)


# speedup vs baseline: 2.7417x; 1.0007x over previous
import functools

import jax
import jax.numpy as jnp
from jax import lax
from jax.experimental import pallas as pl
from jax.experimental.pallas import tpu as pltpu
from jax.experimental.pallas import tpu_sc as plsc

B = 512
D = 128
N_ATOMS = 10000
N_BOND_ROWS = 320000
N_BONDS = N_BOND_ROWS // 2

NC = 2
NS = 16
NW = NC * NS

CHUNK = 128
ATOM_PT = 384
BOND_PT = 5120
ATOM_TOT = ATOM_PT * NW
BOND_TOT = BOND_PT * NW
NA_CHUNKS = ATOM_PT // CHUNK
NB_CHUNKS = BOND_PT // CHUNK

DUMMY = B
ACC_ROWS = 528

_mesh = plsc.VectorSubcoreMesh(core_axis_name="c", subcore_axis_name="s")


@functools.partial(
    pl.kernel,
    out_type=[
        jax.ShapeDtypeStruct((NC, B, D), jnp.float32),
        jax.ShapeDtypeStruct((NC, B, D), jnp.float32),
    ],
    mesh=_mesh,
    scratch_types=[
        pltpu.VMEM((NA_CHUNKS, CHUNK), jnp.int32),
        pltpu.VMEM((NA_CHUNKS, CHUNK), jnp.int32),
        pltpu.VMEM((NB_CHUNKS, CHUNK), jnp.int32),
        pltpu.VMEM((NB_CHUNKS, CHUNK), jnp.int32),
        pltpu.VMEM((CHUNK, D), jnp.float32),
        pltpu.VMEM((CHUNK, D), jnp.float32),
        pltpu.VMEM((33, D), jnp.float32),
        pltpu.VMEM_SHARED((ACC_ROWS, D), jnp.float32),
        pltpu.VMEM_SHARED((ACC_ROWS, D), jnp.float32),
        pltpu.SemaphoreType.DMA,
        pltpu.SemaphoreType.DMA,
    ],
)
def _sc_pool(atom_hbm, bond_hbm, aid_hbm, bid_hbm, pa_hbm, pb_hbm,
             aidx_v, aids_v, bidx_v, bids_v, buf0, buf1, zbuf, acc_a, acc_b,
             semA, semB):
    cid = lax.axis_index("c")
    sid = lax.axis_index("s")
    wid = cid * NS + sid

    zvec = jnp.zeros((16,), jnp.float32)
    for r in range(33):
        for g in range(D // 16):
            zbuf[r, pl.ds(g * 16, 16)] = zvec
    pltpu.sync_copy(zbuf, acc_a.at[pl.ds(sid * 33, 33)])
    pltpu.sync_copy(zbuf, acc_b.at[pl.ds(sid * 33, 33)])

    pltpu.sync_copy(aid_hbm.at[wid], aids_v)
    pltpu.sync_copy(bid_hbm.at[wid], bids_v)

    iota = lax.iota(jnp.int32, 16)
    abase = wid * ATOM_PT
    bbase = wid * BOND_PT

    def build_a(j, carry):
        for g in range(CHUNK // 16):
            aidx_v[j, pl.ds(g * 16, 16)] = jnp.minimum(
                abase + j * CHUNK + g * 16 + iota, N_ATOMS - 1)
        return carry

    def build_b(j, carry):
        for g in range(CHUNK // 16):
            bidx_v[j, pl.ds(g * 16, 16)] = jnp.minimum(
                (bbase + j * CHUNK + g * 16 + iota) * 2, N_BOND_ROWS - 2)
        return carry

    lax.fori_loop(0, NA_CHUNKS, build_a, 0)
    lax.fori_loop(0, NB_CHUNKS, build_b, 0)

    plsc.subcore_barrier()

    a0 = pltpu.async_copy(atom_hbm.at[aidx_v.at[0]], buf0, semA)
    a1 = pltpu.async_copy(atom_hbm.at[aidx_v.at[1]], buf1, semB)
    a0.wait()
    pltpu.sync_copy(buf0, acc_a.at[aids_v.at[0]], add=True)
    a2 = pltpu.async_copy(atom_hbm.at[aidx_v.at[2]], buf0, semA)
    a1.wait()
    pltpu.sync_copy(buf1, acc_a.at[aids_v.at[1]], add=True)
    b1 = pltpu.async_copy(bond_hbm.at[bidx_v.at[1]], buf1, semB)
    a2.wait()
    pltpu.sync_copy(buf0, acc_a.at[aids_v.at[2]], add=True)
    pltpu.async_copy(bond_hbm.at[bidx_v.at[0]], buf0, semA)

    def bond_pair(p, carry):
        j = 2 * p
        pltpu.make_async_copy(bond_hbm.at[bidx_v.at[j]], buf0, semA).wait()
        pltpu.sync_copy(buf0, acc_b.at[bids_v.at[j]], add=True)
        pltpu.async_copy(bond_hbm.at[bidx_v.at[j + 2]], buf0, semA)
        pltpu.make_async_copy(bond_hbm.at[bidx_v.at[j + 1]], buf1, semB).wait()
        pltpu.sync_copy(buf1, acc_b.at[bids_v.at[j + 1]], add=True)
        pltpu.async_copy(bond_hbm.at[bidx_v.at[j + 3]], buf1, semB)
        return carry

    lax.fori_loop(0, NB_CHUNKS // 2 - 1, bond_pair, 0)
    j = NB_CHUNKS - 2
    pltpu.make_async_copy(bond_hbm.at[bidx_v.at[j]], buf0, semA).wait()
    pltpu.sync_copy(buf0, acc_b.at[bids_v.at[j]], add=True)
    pltpu.make_async_copy(bond_hbm.at[bidx_v.at[j + 1]], buf1, semB).wait()
    pltpu.sync_copy(buf1, acc_b.at[bids_v.at[j + 1]], add=True)

    plsc.subcore_barrier()

    pltpu.sync_copy(acc_a.at[pl.ds(sid * 32, 32)], zbuf.at[pl.ds(0, 32)])
    pltpu.sync_copy(zbuf.at[pl.ds(0, 32)], pa_hbm.at[cid, pl.ds(sid * 32, 32)])
    pltpu.sync_copy(acc_b.at[pl.ds(sid * 32, 32)], zbuf.at[pl.ds(0, 32)])
    pltpu.sync_copy(zbuf.at[pl.ds(0, 32)], pb_hbm.at[cid, pl.ds(sid * 32, 32)])


def _combine_body(pa_ref, pb_ref, g_ref, o_ref):
    o_ref[:, 0:D] = pa_ref[0] + pa_ref[1]
    o_ref[:, D:2 * D] = pb_ref[0] + pb_ref[1]
    o_ref[:, 2 * D:3 * D] = g_ref[:]


_combine = pl.pallas_call(
    _combine_body,
    out_shape=jax.ShapeDtypeStruct((B, 3 * D), jnp.float32),
)


def kernel(atom_feats, bond_feats, global_feats, atom_segment_ids, bond_segment_ids):
    aid = atom_segment_ids.astype(jnp.int32)
    bid = bond_segment_ids.astype(jnp.int32)
    apad = DUMMY + (jnp.arange(ATOM_TOT - N_ATOMS, dtype=jnp.int32) % 16)
    bpad = DUMMY + (jnp.arange(BOND_TOT - N_BONDS, dtype=jnp.int32) % 16)
    aid_p = jnp.concatenate([aid, apad]).reshape(NW, NA_CHUNKS, CHUNK)
    bid_p = jnp.concatenate([bid, bpad]).reshape(NW, NB_CHUNKS, CHUNK)
    pa, pb = _sc_pool(atom_feats, bond_feats, aid_p, bid_p)
    return _combine(pa, pb, global_feats)


# speedup vs baseline: 3.3359x; 1.2167x over previous
import functools

import jax
import jax.numpy as jnp
from jax import lax
from jax.experimental import pallas as pl
from jax.experimental.pallas import tpu as pltpu
from jax.experimental.pallas import tpu_sc as plsc

B = 512
D = 128
N_ATOMS = 10000
N_BOND_ROWS = 320000
N_BONDS = N_BOND_ROWS // 2

NC = 2
NS = 16
NW = NC * NS

CHUNK = 128
ATOM_PT = 384
BOND_PT = 5120
NA_CHUNKS = ATOM_PT // CHUNK
NB_CHUNKS = BOND_PT // CHUNK

ATOM_LAST = N_ATOMS - CHUNK
BOND_LAST = N_BONDS - CHUNK

DUMMY = B
ACC_ROWS = 528

_mesh = plsc.VectorSubcoreMesh(core_axis_name="c", subcore_axis_name="s")


@functools.partial(
    pl.kernel,
    out_type=[
        jax.ShapeDtypeStruct((NC, B, D), jnp.float32),
        jax.ShapeDtypeStruct((NC, B, D), jnp.float32),
    ],
    mesh=_mesh,
    scratch_types=[
        pltpu.VMEM((NA_CHUNKS, CHUNK), jnp.int32),
        pltpu.VMEM((NB_CHUNKS, CHUNK), jnp.int32),
        pltpu.VMEM((CHUNK, D), jnp.float32),
        pltpu.VMEM((CHUNK, D), jnp.float32),
        pltpu.VMEM((33, D), jnp.float32),
        pltpu.VMEM_SHARED((ACC_ROWS, D), jnp.float32),
        pltpu.VMEM_SHARED((ACC_ROWS, D), jnp.float32),
        pltpu.SemaphoreType.DMA,
        pltpu.SemaphoreType.DMA,
    ],
)
def _sc_pool(atom_hbm, bond_hbm, aid_hbm, bid_hbm, pa_hbm, pb_hbm,
             aids_v, bids_v, buf0, buf1, zbuf, acc_a, acc_b, semA, semB):
    cid = lax.axis_index("c")
    sid = lax.axis_index("s")
    wid = cid * NS + sid

    zvec = jnp.zeros((16,), jnp.float32)
    for r in range(33):
        for g in range(D // 16):
            zbuf[r, pl.ds(g * 16, 16)] = zvec
    pltpu.sync_copy(zbuf, acc_a.at[pl.ds(sid * 33, 33)])
    pltpu.sync_copy(zbuf, acc_b.at[pl.ds(sid * 33, 33)])

    pltpu.sync_copy(aid_hbm.at[wid], aids_v)
    pltpu.sync_copy(bid_hbm.at[wid], bids_v)

    def astart(j):
        return pl.multiple_of(jnp.minimum(wid * ATOM_PT + j * CHUNK, ATOM_LAST), 8)

    def bstart(j):
        return pl.multiple_of(jnp.minimum(wid * BOND_PT + j * CHUNK, BOND_LAST), 8)

    plsc.subcore_barrier()

    a0 = pltpu.async_copy(atom_hbm.at[pl.ds(astart(0), CHUNK)], buf0, semA)
    a1 = pltpu.async_copy(atom_hbm.at[pl.ds(astart(1), CHUNK)], buf1, semB)
    a0.wait()
    pltpu.sync_copy(buf0, acc_a.at[aids_v.at[0]], add=True)
    a2 = pltpu.async_copy(atom_hbm.at[pl.ds(astart(2), CHUNK)], buf0, semA)
    a1.wait()
    pltpu.sync_copy(buf1, acc_a.at[aids_v.at[1]], add=True)
    b1 = pltpu.async_copy(bond_hbm.at[pl.ds(bstart(1), CHUNK), pl.ds(0, D)],
                          buf1, semB)
    a2.wait()
    pltpu.sync_copy(buf0, acc_a.at[aids_v.at[2]], add=True)
    pltpu.async_copy(bond_hbm.at[pl.ds(bstart(0), CHUNK), pl.ds(0, D)],
                     buf0, semA)

    def bond_pair(p, carry):
        j = 2 * p
        pltpu.make_async_copy(
            bond_hbm.at[pl.ds(bstart(j), CHUNK), pl.ds(0, D)], buf0, semA).wait()
        pltpu.sync_copy(buf0, acc_b.at[bids_v.at[j]], add=True)
        pltpu.async_copy(bond_hbm.at[pl.ds(bstart(j + 2), CHUNK), pl.ds(0, D)],
                         buf0, semA)
        pltpu.make_async_copy(
            bond_hbm.at[pl.ds(bstart(j + 1), CHUNK), pl.ds(0, D)], buf1, semB).wait()
        pltpu.sync_copy(buf1, acc_b.at[bids_v.at[j + 1]], add=True)
        pltpu.async_copy(bond_hbm.at[pl.ds(bstart(j + 3), CHUNK), pl.ds(0, D)],
                         buf1, semB)
        return carry

    lax.fori_loop(0, NB_CHUNKS // 2 - 1, bond_pair, 0)
    j = NB_CHUNKS - 2
    pltpu.make_async_copy(
        bond_hbm.at[pl.ds(bstart(j), CHUNK), pl.ds(0, D)], buf0, semA).wait()
    pltpu.sync_copy(buf0, acc_b.at[bids_v.at[j]], add=True)
    pltpu.make_async_copy(
        bond_hbm.at[pl.ds(bstart(j + 1), CHUNK), pl.ds(0, D)], buf1, semB).wait()
    pltpu.sync_copy(buf1, acc_b.at[bids_v.at[j + 1]], add=True)

    plsc.subcore_barrier()

    pltpu.sync_copy(acc_a.at[pl.ds(sid * 32, 32)], zbuf.at[pl.ds(0, 32)])
    pltpu.sync_copy(zbuf.at[pl.ds(0, 32)], pa_hbm.at[cid, pl.ds(sid * 32, 32)])
    pltpu.sync_copy(acc_b.at[pl.ds(sid * 32, 32)], zbuf.at[pl.ds(0, 32)])
    pltpu.sync_copy(zbuf.at[pl.ds(0, 32)], pb_hbm.at[cid, pl.ds(sid * 32, 32)])


def _combine_body(pa_ref, pb_ref, g_ref, o_ref):
    o_ref[:, 0:D] = pa_ref[0] + pa_ref[1]
    o_ref[:, D:2 * D] = pb_ref[0] + pb_ref[1]
    o_ref[:, 2 * D:3 * D] = g_ref[:]


_combine = pl.pallas_call(
    _combine_body,
    out_shape=jax.ShapeDtypeStruct((B, 3 * D), jnp.float32),
)


def _laid_out_ids(ids, n_rows, n_chunks, last_start):
    c = jnp.arange(NW * n_chunks, dtype=jnp.int32)
    k = jnp.arange(CHUNK, dtype=jnp.int32)
    read_row = jnp.minimum(c * CHUNK, last_start)[:, None] + k[None, :]
    prev_end = jnp.minimum(c * CHUNK, n_rows)[:, None]
    dummy = DUMMY + (read_row % 16)
    laid = jnp.where(read_row >= prev_end,
                     ids[jnp.clip(read_row, 0, n_rows - 1)], dummy)
    return laid.reshape(NW, n_chunks, CHUNK)


def kernel(atom_feats, bond_feats, global_feats, atom_segment_ids, bond_segment_ids):
    aid = atom_segment_ids.astype(jnp.int32)
    bid = bond_segment_ids.astype(jnp.int32)
    aid_p = _laid_out_ids(aid, N_ATOMS, NA_CHUNKS, ATOM_LAST)
    bid_p = _laid_out_ids(bid, N_BONDS, NB_CHUNKS, BOND_LAST)
    bond2 = bond_feats.reshape(N_BONDS, 2 * D)
    pa, pb = _sc_pool(atom_feats, bond2, aid_p, bid_p)
    return _combine(pa, pb, global_feats)
